# Initial kernel scaffold; baseline (speedup 1.0000x reference)
import jax
import jax.numpy as jnp
from jax.experimental import pallas as pl


def kernel(graph, batch_indices):
    raise NotImplementedError("write your pallas kernel here")


# speedup vs baseline: 5.9280x; 5.9280x over previous
import jax
import jax.numpy as jnp
from jax import lax
from jax.experimental import pallas as pl
from jax.experimental.pallas import tpu as pltpu
from jax.experimental.pallas import tpu_sc as plsc

_N = 10000
_D = 128
_B = 64
_ROW = 4 * _D + 16
_ACC = _B * _ROW
_NW = 32
_CHUNK = 320
_TAIL = _N - (_NW - 1) * _CHUNK


def _sc_moments(graph_hbm, idx_hbm, part_hbm, chunk_v, idx_v, acc_v):
    wid = lax.axis_index("s") * 2 + lax.axis_index("c")
    base = wid * _CHUNK

    zeros16 = jnp.zeros((16,), jnp.float32)

    def zbody(j, carry):
        acc_v[pl.ds(j * 16, 16)] = zeros16
        return carry

    lax.fori_loop(0, _ACC // 16, zbody, 0)

    @pl.when(wid < _NW - 1)
    def _():
        pltpu.sync_copy(graph_hbm.at[pl.ds(base * _D, _CHUNK * _D)], chunk_v)
        pltpu.sync_copy(idx_hbm.at[pl.ds(base, _CHUNK)],
                        idx_v.at[pl.ds(0, _CHUNK)])

    @pl.when(wid == _NW - 1)
    def _():
        pltpu.sync_copy(graph_hbm.at[pl.ds(base * _D, _TAIL * _D)],
                        chunk_v.at[pl.ds(0, _TAIL * _D)])
        pltpu.sync_copy(idx_hbm.at[pl.ds(base, _TAIL)],
                        idx_v.at[pl.ds(0, _TAIL)])

    rows = jnp.where(wid == _NW - 1, _TAIL, _CHUNK)
    ones16 = jnp.ones((16,), jnp.float32)

    def body(i, carry):
        s = idx_v[pl.ds(i, 16)][0]
        rb = s * _ROW
        ib = i * _D
        for g in range(_D // 16):
            x = chunk_v[pl.ds(ib + g * 16, 16)]
            x2 = x * x
            plsc.addupdate(acc_v.at[pl.ds(rb + g * 16, 16)], x)
            plsc.addupdate(acc_v.at[pl.ds(rb + _D + g * 16, 16)], x2)
            plsc.addupdate(acc_v.at[pl.ds(rb + 2 * _D + g * 16, 16)], x2 * x)
            plsc.addupdate(acc_v.at[pl.ds(rb + 3 * _D + g * 16, 16)], x2 * x2)
        plsc.addupdate(acc_v.at[pl.ds(rb + 4 * _D, 16)], ones16)
        return carry

    lax.fori_loop(0, rows, body, 0)

    pltpu.sync_copy(acc_v, part_hbm.at[pl.ds(wid * _ACC, _ACC)])


def _finalize(part_ref, out_ref):
    tot = jnp.sum(part_ref[...], axis=0)
    s1 = tot[:, 0:_D]
    s2 = tot[:, _D:2 * _D]
    s3 = tot[:, 2 * _D:3 * _D]
    s4 = tot[:, 3 * _D:4 * _D]
    cnt = tot[:, 4 * _D:4 * _D + 1]
    m = s1 / cnt
    m2 = s2 / cnt
    m3 = s3 / cnt
    m4 = s4 / cnt
    mm = m * m
    var = m2 - mm
    skew = m3 - 3.0 * m * m2 + 2.0 * m * mm
    kurt = m4 - 4.0 * m * m3 + 6.0 * mm * m2 - 3.0 * mm * mm - 3.0
    inf_val = 1000000000000000.0
    skew = jnp.where(skew > inf_val, 0.0, skew)
    skew = jnp.where(jnp.isnan(skew), 0.0, skew)
    kurt = jnp.where(kurt > inf_val, -3.0, kurt)
    kurt = jnp.where(jnp.isnan(kurt), -3.0, kurt)
    out_ref[...] = jnp.concatenate([m, var, skew, kurt], axis=1)


def kernel(graph, batch_indices):
    graph1d = jnp.reshape(graph, (-1,))
    part = pl.kernel(
        _sc_moments,
        out_type=jax.ShapeDtypeStruct((_NW * _ACC,), jnp.float32),
        mesh=plsc.VectorSubcoreMesh(core_axis_name="c", subcore_axis_name="s"),
        scratch_types=[
            pltpu.VMEM((_CHUNK * _D,), jnp.float32),
            pltpu.VMEM((_CHUNK + 16,), jnp.int32),
            pltpu.VMEM((_ACC,), jnp.float32),
        ],
    )(graph1d, batch_indices)
    part3 = jnp.reshape(part, (_NW, _B, _ROW))
    return pl.pallas_call(
        _finalize,
        out_shape=jax.ShapeDtypeStruct((_B, 4 * _D), jnp.float32),
    )(part3)


# speedup vs baseline: 8.3169x; 1.4030x over previous
import jax
import jax.numpy as jnp
from jax import lax
from jax.experimental import pallas as pl
from jax.experimental.pallas import tpu as pltpu
from jax.experimental.pallas import tpu_sc as plsc

_N = 10000
_D = 128
_B = 64
_ROW = 640
_ACC = _B * _ROW
_NW = 32
_CHUNK = 320
_TAIL = _N - (_NW - 1) * _CHUNK
_G = 16


def _sc_moments(graph_hbm, idx_hbm, part_hbm, chunk_v, idx_v, acc_v):
    wid = lax.axis_index("s") * 2 + lax.axis_index("c")
    base = wid * _CHUNK

    zeros16 = jnp.zeros((16,), jnp.float32)

    def zbody(j, carry):
        for k in range(8):
            acc_v[pl.ds(j * 128 + k * 16, 16)] = zeros16
        return carry

    lax.fori_loop(0, _ACC // 128, zbody, 0)

    @pl.when(wid < _NW - 1)
    def _():
        pltpu.sync_copy(graph_hbm.at[pl.ds(base * _D, _CHUNK * _D)], chunk_v)
        pltpu.sync_copy(idx_hbm.at[pl.ds(base, _CHUNK)],
                        idx_v.at[pl.ds(0, _CHUNK)])

    @pl.when(wid == _NW - 1)
    def _():
        pltpu.sync_copy(graph_hbm.at[pl.ds(base * _D, _TAIL * _D)],
                        chunk_v.at[pl.ds(0, _TAIL * _D)])
        pltpu.sync_copy(idx_hbm.at[pl.ds(base, _TAIL)],
                        idx_v.at[pl.ds(0, _TAIL)])

    ngroups = jnp.where(wid == _NW - 1, _TAIL // _G, _CHUNK // _G)
    ones16 = jnp.ones((16,), jnp.float32)

    def body(gi, carry):
        i0 = gi * _G
        idxv = idx_v[pl.ds(i0, _G)]
        s_first = idxv[0]
        s_last = idxv[_G - 1]
        ib = i0 * _D

        @pl.when(s_first == s_last)
        def _():
            rb = s_first * _ROW
            for g in range(_D // 16):
                a1 = zeros16
                a2 = zeros16
                a3 = zeros16
                a4 = zeros16
                for j in range(_G):
                    x = chunk_v[pl.ds(ib + j * _D + g * 16, 16)]
                    x2 = x * x
                    a1 = a1 + x
                    a2 = a2 + x2
                    a3 = a3 + x2 * x
                    a4 = a4 + x2 * x2
                plsc.addupdate(acc_v.at[pl.ds(rb + g * 16, 16)], a1)
                plsc.addupdate(acc_v.at[pl.ds(rb + _D + g * 16, 16)], a2)
                plsc.addupdate(acc_v.at[pl.ds(rb + 2 * _D + g * 16, 16)], a3)
                plsc.addupdate(acc_v.at[pl.ds(rb + 3 * _D + g * 16, 16)], a4)
            plsc.addupdate(acc_v.at[pl.ds(rb + 4 * _D, 16)], ones16 * float(_G))

        @pl.when(s_first != s_last)
        def _():
            for j in range(_G):
                s = idxv[j]
                rb = s * _ROW
                jb = ib + j * _D
                for g in range(_D // 16):
                    x = chunk_v[pl.ds(jb + g * 16, 16)]
                    x2 = x * x
                    plsc.addupdate(acc_v.at[pl.ds(rb + g * 16, 16)], x)
                    plsc.addupdate(acc_v.at[pl.ds(rb + _D + g * 16, 16)], x2)
                    plsc.addupdate(acc_v.at[pl.ds(rb + 2 * _D + g * 16, 16)],
                                   x2 * x)
                    plsc.addupdate(acc_v.at[pl.ds(rb + 3 * _D + g * 16, 16)],
                                   x2 * x2)
                plsc.addupdate(acc_v.at[pl.ds(rb + 4 * _D, 16)], ones16)

        return carry

    lax.fori_loop(0, ngroups, body, 0)

    pltpu.sync_copy(acc_v, part_hbm.at[pl.ds(wid * _ACC, _ACC)])


def _finalize(part_ref, out_ref):
    tot = jnp.sum(part_ref[...], axis=0)
    s1 = tot[:, 0:_D]
    s2 = tot[:, _D:2 * _D]
    s3 = tot[:, 2 * _D:3 * _D]
    s4 = tot[:, 3 * _D:4 * _D]
    cnt = tot[:, 4 * _D:4 * _D + 1]
    m = s1 / cnt
    m2 = s2 / cnt
    m3 = s3 / cnt
    m4 = s4 / cnt
    mm = m * m
    var = m2 - mm
    skew = m3 - 3.0 * m * m2 + 2.0 * m * mm
    kurt = m4 - 4.0 * m * m3 + 6.0 * mm * m2 - 3.0 * mm * mm - 3.0
    inf_val = 1000000000000000.0
    skew = jnp.where(skew > inf_val, 0.0, skew)
    skew = jnp.where(jnp.isnan(skew), 0.0, skew)
    kurt = jnp.where(kurt > inf_val, -3.0, kurt)
    kurt = jnp.where(jnp.isnan(kurt), -3.0, kurt)
    out_ref[...] = jnp.concatenate([m, var, skew, kurt], axis=1)


def kernel(graph, batch_indices):
    graph1d = jnp.reshape(graph, (-1,))
    part = pl.kernel(
        _sc_moments,
        out_type=jax.ShapeDtypeStruct((_NW * _ACC,), jnp.float32),
        mesh=plsc.VectorSubcoreMesh(core_axis_name="c", subcore_axis_name="s"),
        scratch_types=[
            pltpu.VMEM((_CHUNK * _D,), jnp.float32),
            pltpu.VMEM((_CHUNK + 16,), jnp.int32),
            pltpu.VMEM((_ACC,), jnp.float32),
        ],
    )(graph1d, batch_indices)
    part3 = jnp.reshape(part, (_NW, _B, _ROW))
    return pl.pallas_call(
        _finalize,
        out_shape=jax.ShapeDtypeStruct((_B, 4 * _D), jnp.float32),
    )(part3)
